# Initial kernel scaffold; baseline (speedup 1.0000x reference)
import jax
import jax.numpy as jnp
from jax.experimental import pallas as pl


def kernel(ids, weight):
    raise NotImplementedError("write your pallas kernel here")


# speedup vs baseline: 1.0799x; 1.0799x over previous
import functools

import jax
import jax.numpy as jnp
from jax import lax
from jax.experimental import pallas as pl
from jax.experimental.pallas import tpu as pltpu
from jax.experimental.pallas import tpu_sc as plsc

_NC = 2
_NS = 16
_NW = _NC * _NS

_CH = 128
_NBUF = 2


def _embed_lookup(idx2, weight, n_flat, dim):
    nch_total = idx2.shape[0]
    nch = nch_total // _NW

    mesh = plsc.VectorSubcoreMesh(core_axis_name="c", subcore_axis_name="s")

    @functools.partial(
        pl.kernel,
        out_type=jax.ShapeDtypeStruct((n_flat, dim), jnp.float32),
        mesh=mesh,
        compiler_params=pltpu.CompilerParams(use_tc_tiling_on_sc=False),
        scratch_types=[
            pltpu.VMEM((nch, _CH), jnp.int32),
            pltpu.VMEM((_NBUF, _CH, dim), jnp.float32),
            pltpu.SemaphoreType.DMA((_NBUF,)),
        ],
    )
    def k(table_hbm, idx_hbm, out_hbm, idx_v, rows_v, sems):
        wid = lax.axis_index("s") * _NC + lax.axis_index("c")
        row0 = wid * nch

        pltpu.sync_copy(idx_hbm.at[pl.ds(row0, nch)], idx_v)

        for b in range(_NBUF):
            pltpu.async_copy(
                table_hbm.at[idx_v.at[b]], rows_v.at[b], sems.at[b]
            )

        def body(g, carry):
            for b in range(_NBUF):
                j = g * _NBUF + b
                pltpu.make_async_copy(
                    table_hbm.at[idx_v.at[0]], rows_v.at[b], sems.at[b]
                ).wait()
                pltpu.sync_copy(
                    rows_v.at[b],
                    out_hbm.at[pl.ds((row0 + j) * _CH, _CH)],
                )
                nxt = j + _NBUF

                @pl.when(nxt < nch)
                def _():
                    pltpu.async_copy(
                        table_hbm.at[idx_v.at[nxt]], rows_v.at[b], sems.at[b]
                    )

            return carry

        lax.fori_loop(0, nch // _NBUF, body, 0)

    return k(weight, idx2)


def kernel(ids, weight):
    batch, hist = ids.shape
    vocab, dim = weight.shape
    n_flat = batch * hist
    idx2 = ids.reshape(n_flat // _CH, _CH).astype(jnp.int32)
    out = _embed_lookup(idx2, weight, n_flat, dim)
    return out.reshape(batch, hist, dim)


# speedup vs baseline: 1.4493x; 1.3421x over previous
import functools

import jax
import jax.numpy as jnp
from jax import lax
from jax.experimental import pallas as pl
from jax.experimental.pallas import tpu as pltpu
from jax.experimental.pallas import tpu_sc as plsc

_NC = 2
_NS = 16
_NW = _NC * _NS

_CH = 128
_NBUF = 2


def _embed_lookup(idx2, weight, batch, hist, dim):
    nch_total = idx2.shape[0]
    nch = nch_total // _NW
    nd8 = dim // 8
    cpj = batch // _CH

    mesh = plsc.VectorSubcoreMesh(core_axis_name="c", subcore_axis_name="s")

    @functools.partial(
        pl.kernel,
        out_type=jax.ShapeDtypeStruct((hist, nd8, cpj, 8, _CH), jnp.float32),
        mesh=mesh,
        compiler_params=pltpu.CompilerParams(
            use_tc_tiling_on_sc=False, needs_layout_passes=False
        ),
        scratch_types=[
            pltpu.VMEM((nch, _CH), jnp.int32),
            pltpu.VMEM((_NBUF, _CH, dim), jnp.float32),
            pltpu.VMEM((nd8, 8, _CH), jnp.float32),
            pltpu.SemaphoreType.DMA((_NBUF,)),
        ],
    )
    def k(table_hbm, idx_hbm, out_hbm, idx_v, rows_v, tbuf_v, sems):
        wid = lax.axis_index("s") * _NC + lax.axis_index("c")
        row0 = wid * nch

        pltpu.sync_copy(idx_hbm.at[pl.ds(row0, nch)], idx_v)

        for b in range(_NBUF):
            pltpu.async_copy(
                table_hbm.at[idx_v.at[b]], rows_v.at[b], sems.at[b]
            )

        iota16 = lax.iota(jnp.int32, 16)

        def body(g, carry):
            for b in range(_NBUF):
                jj = g * _NBUF + b
                j = row0 + jj
                h = j // cpj
                tj = j - h * cpj
                pltpu.make_async_copy(
                    table_hbm.at[idx_v.at[0]], rows_v.at[b], sems.at[b]
                ).wait()
                for d in range(dim):
                    col = jnp.full((16,), d, dtype=jnp.int32)
                    for qb in range(0, _CH, 16):
                        v = plsc.load_gather(
                            rows_v.at[b], [iota16 + qb, col]
                        )
                        tbuf_v[d // 8, d % 8, pl.ds(qb, 16)] = v
                nxt = jj + _NBUF

                @pl.when(nxt < nch)
                def _():
                    pltpu.async_copy(
                        table_hbm.at[idx_v.at[nxt]], rows_v.at[b], sems.at[b]
                    )

                for ti in range(nd8):
                    pltpu.sync_copy(tbuf_v.at[ti], out_hbm.at[h, ti, tj])

            return carry

        lax.fori_loop(0, nch // _NBUF, body, 0)

    return k(weight, idx2)


def kernel(ids, weight):
    batch, hist = ids.shape
    vocab, dim = weight.shape
    n_flat = batch * hist
    idx2 = ids.T.reshape(n_flat // _CH, _CH).astype(jnp.int32)
    w1 = lax.optimization_barrier(weight.reshape(vocab * dim))
    w2 = w1.reshape(vocab, dim)
    o5 = _embed_lookup(idx2, w2, batch, hist, dim)
    return o5.transpose(2, 4, 0, 1, 3).reshape(batch, hist, dim)


# speedup vs baseline: 1.9601x; 1.3525x over previous
import functools

import jax
import jax.numpy as jnp
from jax import lax
from jax.experimental import pallas as pl
from jax.experimental.pallas import tpu as pltpu
from jax.experimental.pallas import tpu_sc as plsc

_NC = 2
_NS = 16
_NW = _NC * _NS

_CH = 128
_NBUF = 2


def _embed_lookup(idx2, weight, batch, hist, dim):
    nch_total = idx2.shape[0]
    nch = nch_total // _NW
    nd8 = dim // 8
    cpj = batch // _CH

    mesh = plsc.VectorSubcoreMesh(core_axis_name="c", subcore_axis_name="s")

    @functools.partial(
        pl.kernel,
        out_type=jax.ShapeDtypeStruct((hist, nd8, cpj, 8, _CH), jnp.float32),
        mesh=mesh,
        compiler_params=pltpu.CompilerParams(
            use_tc_tiling_on_sc=False, needs_layout_passes=False
        ),
        scratch_types=[
            pltpu.VMEM((nch, _CH), jnp.int32),
            pltpu.VMEM((_NBUF, _CH, dim), jnp.float32),
            pltpu.VMEM((nd8, 8, _CH), jnp.float32),
            pltpu.SemaphoreType.DMA((_NBUF,)),
        ],
    )
    def k(table_hbm, idx_hbm, out_hbm, idx_v, rows_v, tbuf_v, sems):
        wid = lax.axis_index("s") * _NC + lax.axis_index("c")
        row0 = wid * nch

        pltpu.sync_copy(idx_hbm.at[pl.ds(row0, nch)], idx_v)

        for b in range(_NBUF):
            pltpu.async_copy(
                table_hbm.at[idx_v.at[b]], rows_v.at[b], sems.at[b]
            )

        iota16 = lax.iota(jnp.int32, 16)

        def body(g, carry):
            for b in range(_NBUF):
                jj = g * _NBUF + b
                j = row0 + jj
                h = j // cpj
                tj = j - h * cpj
                pltpu.make_async_copy(
                    table_hbm.at[idx_v.at[0]], rows_v.at[b], sems.at[b]
                ).wait()
                for d0 in range(dim):
                    dmv = lax.rem(iota16 + d0, dim)
                    tiv = lax.shift_right_logical(dmv, 3)
                    sv = lax.bitwise_and(dmv, 7)
                    for qb in range(0, _CH, 16):
                        rq = iota16 + qb
                        v = plsc.load_gather(rows_v.at[b], [rq, dmv])
                        plsc.store_scatter(tbuf_v, [tiv, sv, rq], v)
                nxt = jj + _NBUF

                @pl.when(nxt < nch)
                def _():
                    pltpu.async_copy(
                        table_hbm.at[idx_v.at[nxt]], rows_v.at[b], sems.at[b]
                    )

                for ti in range(nd8):
                    pltpu.sync_copy(tbuf_v.at[ti], out_hbm.at[h, ti, tj])

            return carry

        lax.fori_loop(0, nch // _NBUF, body, 0)

    return k(weight, idx2)


def kernel(ids, weight):
    batch, hist = ids.shape
    vocab, dim = weight.shape
    n_flat = batch * hist
    idx2 = ids.T.reshape(n_flat // _CH, _CH).astype(jnp.int32)
    w1 = lax.optimization_barrier(weight.reshape(vocab * dim))
    w2 = w1.reshape(vocab, dim)
    o5 = _embed_lookup(idx2, w2, batch, hist, dim)
    return o5.transpose(2, 4, 0, 1, 3).reshape(batch, hist, dim)


# speedup vs baseline: 2.1882x; 1.1163x over previous
import functools

import jax
import jax.numpy as jnp
from jax import lax
from jax.experimental import pallas as pl
from jax.experimental.pallas import tpu as pltpu
from jax.experimental.pallas import tpu_sc as plsc

_NC = 2
_NS = 16
_NW = _NC * _NS

_CH = 128
_NBUF = 2


def _embed_lookup(idx2, weight, batch, hist, dim):
    nch_total = idx2.shape[0]
    nch = nch_total // _NW
    nd8 = dim // 8
    cpj = batch // _CH

    mesh = plsc.VectorSubcoreMesh(core_axis_name="c", subcore_axis_name="s")

    @functools.partial(
        pl.kernel,
        out_type=jax.ShapeDtypeStruct((hist, nd8, cpj, 8, _CH), jnp.float32),
        mesh=mesh,
        compiler_params=pltpu.CompilerParams(
            use_tc_tiling_on_sc=False, needs_layout_passes=False
        ),
        scratch_types=[
            pltpu.VMEM((nch, _CH), jnp.int32),
            pltpu.VMEM((_NBUF, _CH, dim), jnp.float32),
            pltpu.VMEM((nd8, 8, _CH), jnp.float32),
            pltpu.SemaphoreType.DMA((_NBUF,)),
        ],
    )
    def k(table_hbm, idx_hbm, out_hbm, idx_v, rows_v, tbuf_v, sems):
        wid = lax.axis_index("s") * _NC + lax.axis_index("c")
        row0 = wid * nch

        pltpu.sync_copy(idx_hbm.at[pl.ds(row0, nch)], idx_v)

        for b in range(_NBUF):
            pltpu.async_copy(
                table_hbm.at[idx_v.at[b]], rows_v.at[b], sems.at[b]
            )

        iota16 = lax.iota(jnp.int32, 16)

        def body(g, carry):
            for b in range(_NBUF):
                jj = g * _NBUF + b
                j = row0 + jj
                h = j // cpj
                tj = j - h * cpj
                pltpu.make_async_copy(
                    table_hbm.at[idx_v.at[0]], rows_v.at[b], sems.at[b]
                ).wait()
                for d0 in range(dim):
                    dmv = lax.rem(iota16 + d0, dim)
                    tiv = lax.shift_right_logical(dmv, 3)
                    sv = lax.bitwise_and(dmv, 7)
                    for qb in range(0, _CH, 16):
                        rq = iota16 + qb
                        v = plsc.load_gather(rows_v.at[b], [rq, dmv])
                        plsc.store_scatter(tbuf_v, [tiv, sv, rq], v)
                nxt = jj + _NBUF

                @pl.when(nxt < nch)
                def _():
                    pltpu.async_copy(
                        table_hbm.at[idx_v.at[nxt]], rows_v.at[b], sems.at[b]
                    )

                for ti in range(nd8):
                    pltpu.sync_copy(tbuf_v.at[ti], out_hbm.at[h, ti, tj])

            return carry

        lax.fori_loop(0, nch // _NBUF, body, 0)

    return k(weight, idx2)


def _tc_linearize(w_t, vocab, dim):
    blk = 8192
    g = -(-vocab // blk)
    lines = blk * dim // 128

    def body(x_ref, o_ref):
        x = x_ref[...]
        xt = x.T.reshape(lines, 128 // dim, dim)
        o_ref[...] = jnp.concatenate(
            [xt[:, t, :] for t in range(128 // dim)], axis=1
        )

    return pl.pallas_call(
        body,
        grid=(g,),
        in_specs=[pl.BlockSpec((dim, blk), lambda i: (0, i))],
        out_specs=pl.BlockSpec((lines, 128), lambda i: (i, 0)),
        out_shape=jax.ShapeDtypeStruct((vocab * dim // 128, 128), jnp.float32),
    )(w_t)


def kernel(ids, weight):
    batch, hist = ids.shape
    vocab, dim = weight.shape
    n_flat = batch * hist
    idx2 = ids.T.reshape(n_flat // _CH, _CH).astype(jnp.int32)
    wlin = _tc_linearize(weight.T, vocab, dim)
    w2 = lax.optimization_barrier(wlin).reshape(vocab, dim)
    o5 = _embed_lookup(idx2, w2, batch, hist, dim)
    return o5.transpose(2, 4, 0, 1, 3).reshape(batch, hist, dim)


# speedup vs baseline: 2.3024x; 1.0522x over previous
import functools

import jax
import jax.numpy as jnp
from jax import lax
from jax.experimental import pallas as pl
from jax.experimental.pallas import tpu as pltpu
from jax.experimental.pallas import tpu_sc as plsc

_NC = 2
_NS = 16
_NW = _NC * _NS

_CH = 128
_NBUF = 2
_TBUF = 2


def _embed_lookup(idx2, weight, batch, hist, dim):
    nch_total = idx2.shape[0]
    nch = nch_total // _NW
    nd8 = dim // 8
    cpj = batch // _CH

    mesh = plsc.VectorSubcoreMesh(core_axis_name="c", subcore_axis_name="s")

    @functools.partial(
        pl.kernel,
        out_type=jax.ShapeDtypeStruct((hist, nd8, cpj, 8, _CH), jnp.float32),
        mesh=mesh,
        compiler_params=pltpu.CompilerParams(
            use_tc_tiling_on_sc=False, needs_layout_passes=False
        ),
        scratch_types=[
            pltpu.VMEM((nch, _CH), jnp.int32),
            pltpu.VMEM((_NBUF, _CH, dim), jnp.float32),
            pltpu.VMEM((_TBUF, nd8, 8, _CH), jnp.float32),
            pltpu.SemaphoreType.DMA((_NBUF,)),
            pltpu.SemaphoreType.DMA((_TBUF,)),
        ],
    )
    def k(table_hbm, idx_hbm, out_hbm, idx_v, rows_v, tbuf_v, sems, osems):
        wid = lax.axis_index("s") * _NC + lax.axis_index("c")
        row0 = wid * nch

        pltpu.sync_copy(idx_hbm.at[pl.ds(row0, nch)], idx_v)

        for b in range(_NBUF):
            pltpu.async_copy(
                table_hbm.at[idx_v.at[b]], rows_v.at[b], sems.at[b]
            )

        iota16 = lax.iota(jnp.int32, 16)

        def body(g, carry):
            for b in range(_NBUF):
                jj = g * _NBUF + b
                j = row0 + jj
                h = j // cpj
                tj = j - h * cpj
                tb = b % _TBUF
                pltpu.make_async_copy(
                    table_hbm.at[idx_v.at[0]], rows_v.at[b], sems.at[b]
                ).wait()
                @pl.when(jj >= _TBUF)
                def _():
                    for ti in range(nd8):
                        pltpu.make_async_copy(
                            tbuf_v.at[tb, ti],
                            out_hbm.at[h, ti, tj],
                            osems.at[tb],
                        ).wait()

                for d0 in range(dim):
                    dmv = lax.rem(iota16 + d0, dim)
                    tiv = lax.shift_right_logical(dmv, 3)
                    sv = lax.bitwise_and(dmv, 7)
                    for qb in range(0, _CH, 16):
                        rq = iota16 + qb
                        v = plsc.load_gather(rows_v.at[b], [rq, dmv])
                        plsc.store_scatter(tbuf_v.at[tb], [tiv, sv, rq], v)
                nxt = jj + _NBUF

                @pl.when(nxt < nch)
                def _():
                    pltpu.async_copy(
                        table_hbm.at[idx_v.at[nxt]], rows_v.at[b], sems.at[b]
                    )

                for ti in range(nd8):
                    pltpu.async_copy(
                        tbuf_v.at[tb, ti], out_hbm.at[h, ti, tj], osems.at[tb]
                    )

            return carry

        lax.fori_loop(0, nch // _NBUF, body, 0)

        for tb in range(_TBUF):
            for ti in range(nd8):
                pltpu.make_async_copy(
                    tbuf_v.at[tb, ti], out_hbm.at[0, ti, 0], osems.at[tb]
                ).wait()

    return k(weight, idx2)


def _tc_linearize(w_t, vocab, dim):
    blk = 8192
    g = -(-vocab // blk)
    lines = blk * dim // 128

    def body(x_ref, o_ref):
        x = x_ref[...]
        eye = jnp.eye(dim, dtype=jnp.float32)
        xt = jax.lax.dot_general(
            x, eye, (((0,), (0,)), ((), ())),
            preferred_element_type=jnp.float32,
        )
        xt = xt.reshape(lines, 128 // dim, dim)
        o_ref[...] = jnp.concatenate(
            [xt[:, t, :] for t in range(128 // dim)], axis=1
        )

    return pl.pallas_call(
        body,
        grid=(g,),
        in_specs=[pl.BlockSpec((dim, blk), lambda i: (0, i))],
        out_specs=pl.BlockSpec((lines, 128), lambda i: (i, 0)),
        out_shape=jax.ShapeDtypeStruct((vocab * dim // 128, 128), jnp.float32),
    )(w_t)


def kernel(ids, weight):
    batch, hist = ids.shape
    vocab, dim = weight.shape
    n_flat = batch * hist
    idx2 = ids.T.reshape(n_flat // _CH, _CH).astype(jnp.int32)
    wlin = _tc_linearize(weight.T, vocab, dim)
    w2 = lax.optimization_barrier(wlin).reshape(vocab, dim)
    o5 = _embed_lookup(idx2, w2, batch, hist, dim)
    return o5.transpose(2, 4, 0, 1, 3).reshape(batch, hist, dim)
